# baseline (speedup 1.0000x reference)
import jax
import jax.numpy as jnp
from jax import lax
from jax.experimental import pallas as pl

_N = 4096
_K = 30
_NEG = -1e15
_R = 256


def _topk_softmax_kernel(x_ref, o_ref):
    x = x_ref[...]
    bi = lax.bitcast_convert_type(x, jnp.int32)
    key = bi ^ jnp.bitwise_and(jnp.right_shift(bi, 31), jnp.int32(0x7FFFFFFF))
    min32 = jnp.int32(-(2**31))
    u = jnp.bitwise_xor(key, min32)
    dh = (lax.shift_right_logical(u, 16) - 32768).astype(jnp.int16)
    dl = (jnp.bitwise_and(u, jnp.int32(0xFFFF)) - 32768).astype(jnp.int16)

    def count16(mask):
        ones = jnp.where(mask, jnp.bfloat16(1.0), jnp.bfloat16(0.0))
        part = jnp.sum(ones.reshape(_R, 32, 128), axis=1)
        return jnp.sum(part.astype(jnp.float32), axis=1, keepdims=True)

    def level(g, base_cnt):
        def body(i, wd):
            bit = jnp.left_shift(jnp.int32(1), jnp.int32(15) - i)
            cand = jnp.bitwise_or(wd, bit)
            cand16 = (cand - 32768).astype(jnp.int16)
            cnt = base_cnt + count16(g >= cand16)
            return jnp.where(cnt >= _K, cand, wd)

        return lax.fori_loop(0, 16, body, jnp.zeros((_R, 1), jnp.int32))

    wh = level(dh, jnp.zeros((_R, 1), jnp.float32))
    wh16 = (wh - 32768).astype(jnp.int16)
    hi_cnt = count16(dh > wh16)
    gl = jnp.where(dh == wh16, dl, jnp.int16(-(2**15)))
    wl = level(gl, hi_cnt)

    kth = jnp.bitwise_xor(
        jnp.bitwise_or(jnp.left_shift(wh, 16), wl), min32)
    keep = (key >= kth) & (x > 0.0)
    m = jnp.where(keep, x, _NEG)
    rowmax = jnp.max(m, axis=1, keepdims=True)
    e = jnp.exp(m - rowmax)
    s = jnp.sum(e, axis=1, keepdims=True)
    o_ref[...] = e / s


def kernel(new_supports):
    n = new_supports.shape[0]
    return pl.pallas_call(
        _topk_softmax_kernel,
        grid=(n // _R,),
        in_specs=[pl.BlockSpec((_R, _N), lambda i: (i, 0))],
        out_specs=pl.BlockSpec((_R, _N), lambda i: (i, 0)),
        out_shape=jax.ShapeDtypeStruct((n, _N), jnp.float32),
    )(new_supports)


# speedup vs baseline: 2.7454x; 2.7454x over previous
import jax
import jax.numpy as jnp
from jax.experimental import pallas as pl

_N = 4096
_K = 30
_NEG = -1e15
_ROWS_PER_BLOCK = 256


def _topk_softmax_kernel(x_ref, o_ref):
    x = x_ref[...]
    bi = jax.lax.bitcast_convert_type(x, jnp.int32)
    key = bi ^ jnp.bitwise_and(jnp.right_shift(bi, 31), jnp.int32(0x7FFFFFFF))
    min32 = jnp.int32(-(2**31))

    def body(i, w):
        bit = jnp.left_shift(jnp.int32(1), jnp.int32(31) - i)
        cand_w = jnp.bitwise_or(w, bit)
        cand_t = jnp.bitwise_xor(cand_w, min32)
        cnt = jnp.sum((key >= cand_t).astype(jnp.float32), axis=1, keepdims=True)
        return jnp.where(cnt >= _K, cand_w, w)

    w0 = jnp.zeros((x.shape[0], 1), jnp.int32)
    w = jax.lax.fori_loop(0, 32, body, w0, unroll=4)
    kth = jnp.bitwise_xor(w, min32)

    keep = (key >= kth) & (x > 0.0)
    m = jnp.where(keep, x, _NEG)
    rowmax = jnp.max(m, axis=1, keepdims=True)
    e = jnp.exp(m - rowmax)
    s = jnp.sum(e, axis=1, keepdims=True)
    o_ref[...] = e / s


def kernel(new_supports):
    n = new_supports.shape[0]
    r = _ROWS_PER_BLOCK
    return pl.pallas_call(
        _topk_softmax_kernel,
        grid=(n // r,),
        in_specs=[pl.BlockSpec((r, _N), lambda i: (i, 0))],
        out_specs=pl.BlockSpec((r, _N), lambda i: (i, 0)),
        out_shape=jax.ShapeDtypeStruct((n, _N), jnp.float32),
    )(new_supports)


# speedup vs baseline: 2.8004x; 1.0200x over previous
import jax
import jax.numpy as jnp
from jax.experimental import pallas as pl

_N = 4096
_K = 30
_NEG = -1e15
_ROWS_PER_BLOCK = 256


def _topk_softmax_kernel(x_ref, o_ref):
    x = x_ref[...]
    bi = jax.lax.bitcast_convert_type(x, jnp.int32)
    key = bi ^ jnp.bitwise_and(jnp.right_shift(bi, 31), jnp.int32(0x7FFFFFFF))
    min32 = jnp.int32(-(2**31))

    def body(i, w):
        bit = jnp.left_shift(jnp.int32(1), jnp.int32(31) - i)
        cand_w = jnp.bitwise_or(w, bit)
        cand_t = jnp.bitwise_xor(cand_w, min32)
        cnt = jnp.sum((key >= cand_t).astype(jnp.float32), axis=1, keepdims=True)
        return jnp.where(cnt >= _K, cand_w, w)

    w0 = jnp.zeros((x.shape[0], 1), jnp.int32)
    w = jax.lax.fori_loop(0, 32, body, w0, unroll=8)
    kth = jnp.bitwise_xor(w, min32)

    keep = (key >= kth) & (x > 0.0)
    m = jnp.where(keep, x, _NEG)
    rowmax = jnp.max(m, axis=1, keepdims=True)
    e = jnp.exp(m - rowmax)
    s = jnp.sum(e, axis=1, keepdims=True)
    o_ref[...] = e / s


def kernel(new_supports):
    n = new_supports.shape[0]
    r = _ROWS_PER_BLOCK
    return pl.pallas_call(
        _topk_softmax_kernel,
        grid=(n // r,),
        in_specs=[pl.BlockSpec((r, _N), lambda i: (i, 0))],
        out_specs=pl.BlockSpec((r, _N), lambda i: (i, 0)),
        out_shape=jax.ShapeDtypeStruct((n, _N), jnp.float32),
    )(new_supports)
